# baseline (speedup 1.0000x reference)
import functools

import jax
import jax.numpy as jnp
from jax import lax
from jax.experimental import pallas as pl
from jax.experimental.pallas import tpu as pltpu
from jax.experimental.pallas import tpu_sc as plsc

_N = 10000
_DEG = 32
_E = _N * _DEG
_NC = 128
_DE = 16
_HID = 64

_NUM_WORKERS = 32

_GA_BLK = 128
_GA_NBLOCKS = _E // _GA_BLK
_GA_MAXB = -(-_GA_NBLOCKS // _NUM_WORKERS)


def _gather_wg(weights, idxe2):
    mesh = plsc.VectorSubcoreMesh(core_axis_name="c", subcore_axis_name="s")

    @functools.partial(
        pl.kernel, mesh=mesh,
        out_type=jax.ShapeDtypeStruct((_E, _NC // 2), jnp.int32),
        scratch_types=[
            pltpu.VMEM((_GA_MAXB, 1, _GA_BLK), jnp.int32),
            pltpu.VMEM((_GA_BLK, _NC), jnp.float32),
            pltpu.VMEM((_GA_BLK, _NC), jnp.float32),
            pltpu.VMEM((_GA_BLK, _NC // 2), jnp.int32),
            pltpu.VMEM((_GA_BLK, _NC // 2), jnp.int32),
            pltpu.SemaphoreType.DMA,
            pltpu.SemaphoreType.DMA,
            pltpu.SemaphoreType.DMA,
            pltpu.SemaphoreType.DMA,
        ],
    )
    def k(w_hbm, idxe_hbm, out_hbm, idx_v, r0_v, r1_v, o0_v, o1_v,
          sg0, sg1, so0, so1):
        rows = (r0_v, r1_v)
        obuf = (o0_v, o1_v)
        gsems = (sg0, sg1)
        osems = (so0, so1)
        wid = lax.axis_index("s") * 2 + lax.axis_index("c")
        blo = (wid * _GA_NBLOCKS) // _NUM_WORKERS
        bhi = ((wid + 1) * _GA_NBLOCKS) // _NUM_WORKERS
        nblk = bhi - blo
        pltpu.sync_copy(idxe_hbm.at[pl.ds(blo, _GA_MAXB)], idx_v)

        def fire(i, p):
            pltpu.async_copy(w_hbm.at[idx_v.at[i, 0]], rows[p], gsems[p])

        fire(0, 0)

        @pl.when(nblk > 1)
        def _():
            fire(1, 1)

        def body(it, carry):
            for p in range(2):
                i = it * 2 + p

                @pl.when(i < nblk)
                def _():
                    pltpu.make_async_copy(
                        w_hbm.at[idx_v.at[i, 0]], rows[p], gsems[p]).wait()

                    @pl.when(i >= 2)
                    def _():
                        pltpu.make_async_copy(
                            obuf[p], out_hbm.at[pl.ds(0, _GA_BLK)],
                            osems[p]).wait()

                    def rbody(r, c):
                        for q in range(4):
                            a = rows[p][r, pl.ds(32 * q, 16)]
                            b = rows[p][r, pl.ds(32 * q + 16, 16)]
                            ai = lax.bitcast_convert_type(a, jnp.int32) + 0x8000
                            bi = lax.bitcast_convert_type(b, jnp.int32) + 0x8000
                            lo = lax.shift_right_logical(ai, 16)
                            hi = jnp.bitwise_and(bi, jnp.int32(-65536))
                            obuf[p][r, pl.ds(16 * q, 16)] = lo | hi
                        return c

                    lax.fori_loop(0, _GA_BLK, rbody, 0, unroll=4)
                    pltpu.async_copy(
                        obuf[p],
                        out_hbm.at[pl.ds((blo + i) * _GA_BLK, _GA_BLK)],
                        osems[p])

                    @pl.when(i + 2 < nblk)
                    def _():
                        fire(i + 2, p)

            return carry

        lax.fori_loop(0, (nblk + 1) // 2, body, 0)
        pltpu.make_async_copy(obuf[0], out_hbm.at[pl.ds(0, _GA_BLK)],
                              osems[0]).wait()

        @pl.when(nblk > 1)
        def _():
            pltpu.make_async_copy(obuf[1], out_hbm.at[pl.ds(0, _GA_BLK)],
                                  osems[1]).wait()

    return k(weights, idxe2)


_FB = 2560


def _filter_net(ef, Wf1, bf1, Wf2, bf2):
    def body(ef_ref, w1_ref, b1_ref, w2_ref, b2_ref, out_ref):
        h1 = jnp.dot(ef_ref[...], w1_ref[...],
                     preferred_element_type=jnp.float32) + b1_ref[...]
        h1 = jnp.maximum(h1, 0.0)
        w = jnp.dot(h1, w2_ref[...],
                    preferred_element_type=jnp.float32) + b2_ref[...]
        out_ref[...] = w

    return pl.pallas_call(
        body,
        grid=(_E // _FB,),
        in_specs=[
            pl.BlockSpec((_FB, _DE), lambda i: (i, 0)),
            pl.BlockSpec((_DE, _HID), lambda i: (0, 0)),
            pl.BlockSpec((1, _HID), lambda i: (0, 0)),
            pl.BlockSpec((_HID, _NC), lambda i: (0, 0)),
            pl.BlockSpec((1, _NC), lambda i: (0, 0)),
        ],
        out_specs=pl.BlockSpec((_FB, _NC), lambda i: (i, 0)),
        out_shape=jax.ShapeDtypeStruct((_E, _NC), jnp.float32),
    )(ef, Wf1, bf1.reshape(1, _HID), Wf2, bf2.reshape(1, _NC))


_NBLK = 4
_EBLK = _NBLK * _DEG
_NBLOCKS = _N // _NBLK
_MAXB = -(-_NBLOCKS // _NUM_WORKERS)


def _gconv(h, wg, idxn2):
    mesh = plsc.VectorSubcoreMesh(core_axis_name="c", subcore_axis_name="s")

    @functools.partial(
        pl.kernel, mesh=mesh,
        out_type=jax.ShapeDtypeStruct((_N, _NC), jnp.float32),
        scratch_types=[
            pltpu.VMEM((_MAXB, 1, _EBLK), jnp.int32),
            pltpu.VMEM((_EBLK, _NC), jnp.float32),
            pltpu.VMEM((_EBLK, _NC), jnp.float32),
            pltpu.VMEM((_EBLK, _NC // 2), jnp.int32),
            pltpu.VMEM((_EBLK, _NC // 2), jnp.int32),
            pltpu.VMEM((_NBLK, _NC), jnp.float32),
            pltpu.VMEM((_NBLK, _NC), jnp.float32),
            pltpu.SemaphoreType.DMA,
            pltpu.SemaphoreType.DMA,
            pltpu.SemaphoreType.DMA,
            pltpu.SemaphoreType.DMA,
            pltpu.SemaphoreType.DMA,
            pltpu.SemaphoreType.DMA,
        ],
    )
    def k(h_hbm, wg_hbm, idxn_hbm, out_hbm, idx_v,
          r0_v, r1_v, w0_v, w1_v, o0_v, o1_v,
          sg0, sg1, sw0, sw1, so0, so1):
        wid = lax.axis_index("s") * 2 + lax.axis_index("c")
        blo = (wid * _NBLOCKS) // _NUM_WORKERS
        bhi = ((wid + 1) * _NBLOCKS) // _NUM_WORKERS
        nblk = bhi - blo
        pltpu.sync_copy(idxn_hbm.at[pl.ds(blo, _MAXB)], idx_v)
        rows = (r0_v, r1_v)
        wbuf = (w0_v, w1_v)
        obuf = (o0_v, o1_v)
        gsems = (sg0, sg1)
        wsems = (sw0, sw1)
        osems = (so0, so1)

        def fire(i, p):
            pltpu.async_copy(h_hbm.at[idx_v.at[i, 0]], rows[p], gsems[p])
            pltpu.async_copy(wg_hbm.at[pl.ds((blo + i) * _EBLK, _EBLK)],
                             wbuf[p], wsems[p])

        fire(0, 0)

        @pl.when(nblk > 1)
        def _():
            fire(1, 1)

        def body(it, carry):
            for p in range(2):
                i = it * 2 + p

                @pl.when(i < nblk)
                def _():
                    pltpu.make_async_copy(
                        h_hbm.at[idx_v.at[i, 0]], rows[p], gsems[p]).wait()
                    pltpu.make_async_copy(
                        wg_hbm.at[pl.ds(0, _EBLK)], wbuf[p], wsems[p]).wait()
                    @pl.when(i >= 2)
                    def _():
                        pltpu.make_async_copy(
                            obuf[p], out_hbm.at[pl.ds(0, _NBLK)],
                            osems[p]).wait()

                    for nn in range(_NBLK):
                        def ebody(j, accs):
                            e = nn * _DEG + j
                            accs = list(accs)
                            for q in range(4):
                                w = wbuf[p][e, pl.ds(16 * q, 16)]
                                wa = lax.bitcast_convert_type(
                                    lax.shift_left(w, 16), jnp.float32)
                                wb = lax.bitcast_convert_type(
                                    jnp.bitwise_and(w, jnp.int32(-65536)),
                                    jnp.float32)
                                ha = rows[p][e, pl.ds(32 * q, 16)]
                                hb = rows[p][e, pl.ds(32 * q + 16, 16)]
                                accs[2 * q] = accs[2 * q] + ha * wa
                                accs[2 * q + 1] = accs[2 * q + 1] + hb * wb
                            return tuple(accs)

                        accs = lax.fori_loop(
                            0, _DEG, ebody,
                            tuple(jnp.zeros((16,), jnp.float32)
                                  for _ in range(_NC // 16)),
                            unroll=4)
                        for c in range(_NC // 16):
                            obuf[p][nn, pl.ds(c * 16, 16)] = accs[c]

                    pltpu.async_copy(
                        obuf[p],
                        out_hbm.at[pl.ds((blo + i) * _NBLK, _NBLK)], osems[p])

                    @pl.when(i + 2 < nblk)
                    def _():
                        fire(i + 2, p)

            return carry

        lax.fori_loop(0, (nblk + 1) // 2, body, 0)
        pltpu.make_async_copy(obuf[0], out_hbm.at[pl.ds(0, _NBLK)],
                              osems[0]).wait()

        @pl.when(nblk > 1)
        def _():
            pltpu.make_async_copy(obuf[1], out_hbm.at[pl.ds(0, _NBLK)],
                                  osems[1]).wait()

    return k(h, wg, idxn2)


_GB = 1000


def _gru_body(m_ref, s_ref, d_ref, wih_ref, whh_ref, bih_ref, bhh_ref):
    inv = 1.0 / jnp.maximum(d_ref[...].astype(jnp.float32), 1.0)
    x = m_ref[...] * inv
    gi = jnp.dot(x, wih_ref[...],
                 preferred_element_type=jnp.float32) + bih_ref[...]
    gh = jnp.dot(s_ref[...], whh_ref[...],
                 preferred_element_type=jnp.float32) + bhh_ref[...]
    ir, iz, inn = gi[:, :_NC], gi[:, _NC:2 * _NC], gi[:, 2 * _NC:]
    hr, hz, hn = gh[:, :_NC], gh[:, _NC:2 * _NC], gh[:, 2 * _NC:]
    r = jax.nn.sigmoid(ir + hr)
    z = jax.nn.sigmoid(iz + hz)
    n = jnp.tanh(inn + r * hn)
    return (1.0 - z) * n + z * s_ref[...]


_ROW_SPEC = pl.BlockSpec((_GB, _NC), lambda i: (i, 0))
_GRU_IN_SPECS = [
    _ROW_SPEC,
    _ROW_SPEC,
    pl.BlockSpec((_GB, 1), lambda i: (i, 0)),
    pl.BlockSpec((_NC, 3 * _NC), lambda i: (0, 0)),
    pl.BlockSpec((_NC, 3 * _NC), lambda i: (0, 0)),
    pl.BlockSpec((1, 3 * _NC), lambda i: (0, 0)),
    pl.BlockSpec((1, 3 * _NC), lambda i: (0, 0)),
]


def _gru_plain(m, s, degs2, W_ih, W_hh, b_ih2, b_hh2):
    def body(m_ref, s_ref, d_ref, wih_ref, whh_ref, bih_ref, bhh_ref,
             out_ref):
        out_ref[...] = _gru_body(m_ref, s_ref, d_ref, wih_ref, whh_ref,
                                 bih_ref, bhh_ref)

    return pl.pallas_call(
        body,
        grid=(_N // _GB,),
        in_specs=_GRU_IN_SPECS,
        out_specs=_ROW_SPEC,
        out_shape=jax.ShapeDtypeStruct((_N, _NC), jnp.float32),
    )(m, s, degs2, W_ih, W_hh, b_ih2, b_hh2)


def _gru_skip(m, s, add, degs2, W_ih, W_hh, b_ih2, b_hh2):
    def body(m_ref, s_ref, a_ref, d_ref, wih_ref, whh_ref, bih_ref, bhh_ref,
             raw_ref, out_ref):
        raw = _gru_body(m_ref, s_ref, d_ref, wih_ref, whh_ref, bih_ref,
                        bhh_ref)
        raw_ref[...] = raw
        out_ref[...] = raw + a_ref[...]

    return pl.pallas_call(
        body,
        grid=(_N // _GB,),
        in_specs=[_ROW_SPEC, _ROW_SPEC] + _GRU_IN_SPECS[1:],
        out_specs=[_ROW_SPEC, _ROW_SPEC],
        out_shape=[jax.ShapeDtypeStruct((_N, _NC), jnp.float32),
                   jax.ShapeDtypeStruct((_N, _NC), jnp.float32)],
    )(m, s, add, degs2, W_ih, W_hh, b_ih2, b_hh2)


def kernel(hx, edgefeats, idxn, idxe, degs, Wf1, bf1, Wf2, bf2,
           W_ih, W_hh, b_ih, b_hh):
    weights = _filter_net(edgefeats, Wf1, bf1, Wf2, bf2)
    wg = _gather_wg(weights, idxe.reshape(_GA_NBLOCKS, 1, _GA_BLK))
    idxn2 = idxn.reshape(_NBLOCKS, 1, _EBLK)
    degs2 = degs.reshape(_N, 1)
    bih2 = b_ih.reshape(1, 3 * _NC)
    bhh2 = b_hh.reshape(1, 3 * _NC)

    def g_plain(s):
        m = _gconv(s, wg, idxn2)
        return _gru_plain(m, s, degs2, W_ih, W_hh, bih2, bhh2)

    def g_skip(s, add):
        m = _gconv(s, wg, idxn2)
        return _gru_skip(m, s, add, degs2, W_ih, W_hh, bih2, bhh2)

    s1 = g_plain(hx)
    s2 = g_plain(s1)
    r3, s3 = g_skip(s2, s1)
    s4 = g_plain(s3)
    r5, s5 = g_skip(s4, r3)
    s6 = g_plain(s5)
    r7, s7 = g_skip(s6, r5)
    s8 = g_plain(s7)
    _, s9 = g_skip(s8, r7)
    s10 = g_plain(s9)
    return jnp.concatenate(
        [hx, s1, s2, s3, s4, s5, s6, s7, s8, s9, s10], axis=1)


# speedup vs baseline: 1.0911x; 1.0911x over previous
import functools

import jax
import jax.numpy as jnp
from jax import lax
from jax.experimental import pallas as pl
from jax.experimental.pallas import tpu as pltpu
from jax.experimental.pallas import tpu_sc as plsc

_N = 10000
_DEG = 32
_E = _N * _DEG
_NC = 128
_DE = 16
_HID = 64

_NUM_WORKERS = 32

_GA_BLK = 128
_GA_NBLOCKS = _E // _GA_BLK
_GA_MAXB = -(-_GA_NBLOCKS // _NUM_WORKERS)


def _gather_wg(weights, idxe2):
    mesh = plsc.VectorSubcoreMesh(core_axis_name="c", subcore_axis_name="s")

    @functools.partial(
        pl.kernel, mesh=mesh,
        out_type=jax.ShapeDtypeStruct((_E, _NC // 2), jnp.int32),
        scratch_types=[
            pltpu.VMEM((_GA_MAXB, 1, _GA_BLK), jnp.int32),
            pltpu.VMEM((_GA_BLK, _NC), jnp.float32),
            pltpu.VMEM((_GA_BLK, _NC), jnp.float32),
            pltpu.VMEM((_GA_BLK, _NC // 2), jnp.int32),
            pltpu.VMEM((_GA_BLK, _NC // 2), jnp.int32),
            pltpu.SemaphoreType.DMA,
            pltpu.SemaphoreType.DMA,
            pltpu.SemaphoreType.DMA,
            pltpu.SemaphoreType.DMA,
        ],
    )
    def k(w_hbm, idxe_hbm, out_hbm, idx_v, r0_v, r1_v, o0_v, o1_v,
          sg0, sg1, so0, so1):
        rows = (r0_v, r1_v)
        obuf = (o0_v, o1_v)
        gsems = (sg0, sg1)
        osems = (so0, so1)
        wid = lax.axis_index("s") * 2 + lax.axis_index("c")
        blo = (wid * _GA_NBLOCKS) // _NUM_WORKERS
        bhi = ((wid + 1) * _GA_NBLOCKS) // _NUM_WORKERS
        nblk = bhi - blo
        pltpu.sync_copy(idxe_hbm.at[pl.ds(blo, _GA_MAXB)], idx_v)

        def fire(i, p):
            pltpu.async_copy(w_hbm.at[idx_v.at[i, 0]], rows[p], gsems[p])

        fire(0, 0)

        @pl.when(nblk > 1)
        def _():
            fire(1, 1)

        def body(it, carry):
            for p in range(2):
                i = it * 2 + p

                @pl.when(i < nblk)
                def _():
                    pltpu.make_async_copy(
                        w_hbm.at[idx_v.at[i, 0]], rows[p], gsems[p]).wait()

                    @pl.when(i >= 2)
                    def _():
                        pltpu.make_async_copy(
                            obuf[p], out_hbm.at[pl.ds(0, _GA_BLK)],
                            osems[p]).wait()

                    def rbody(r, c):
                        for q in range(4):
                            a = rows[p][r, pl.ds(32 * q, 16)]
                            b = rows[p][r, pl.ds(32 * q + 16, 16)]
                            ai = lax.bitcast_convert_type(a, jnp.int32) + 0x8000
                            bi = lax.bitcast_convert_type(b, jnp.int32) + 0x8000
                            lo = lax.shift_right_logical(ai, 16)
                            hi = jnp.bitwise_and(bi, jnp.int32(-65536))
                            obuf[p][r, pl.ds(16 * q, 16)] = lo | hi
                        return c

                    lax.fori_loop(0, _GA_BLK, rbody, 0, unroll=4)
                    pltpu.async_copy(
                        obuf[p],
                        out_hbm.at[pl.ds((blo + i) * _GA_BLK, _GA_BLK)],
                        osems[p])

                    @pl.when(i + 2 < nblk)
                    def _():
                        fire(i + 2, p)

            return carry

        lax.fori_loop(0, (nblk + 1) // 2, body, 0)
        pltpu.make_async_copy(obuf[0], out_hbm.at[pl.ds(0, _GA_BLK)],
                              osems[0]).wait()

        @pl.when(nblk > 1)
        def _():
            pltpu.make_async_copy(obuf[1], out_hbm.at[pl.ds(0, _GA_BLK)],
                                  osems[1]).wait()

    return k(weights, idxe2)


_FB = 2560


def _filter_net(ef, Wf1, bf1, Wf2, bf2):
    def body(ef_ref, w1_ref, b1_ref, w2_ref, b2_ref, out_ref):
        h1 = jnp.dot(ef_ref[...], w1_ref[...],
                     preferred_element_type=jnp.float32) + b1_ref[...]
        h1 = jnp.maximum(h1, 0.0)
        w = jnp.dot(h1, w2_ref[...],
                    preferred_element_type=jnp.float32) + b2_ref[...]
        out_ref[...] = w

    return pl.pallas_call(
        body,
        grid=(_E // _FB,),
        in_specs=[
            pl.BlockSpec((_FB, _DE), lambda i: (i, 0)),
            pl.BlockSpec((_DE, _HID), lambda i: (0, 0)),
            pl.BlockSpec((1, _HID), lambda i: (0, 0)),
            pl.BlockSpec((_HID, _NC), lambda i: (0, 0)),
            pl.BlockSpec((1, _NC), lambda i: (0, 0)),
        ],
        out_specs=pl.BlockSpec((_FB, _NC), lambda i: (i, 0)),
        out_shape=jax.ShapeDtypeStruct((_E, _NC), jnp.float32),
    )(ef, Wf1, bf1.reshape(1, _HID), Wf2, bf2.reshape(1, _NC))


_NBLK = 4
_EBLK = _NBLK * _DEG
_NBLOCKS = _N // _NBLK
_MAXB = -(-_NBLOCKS // _NUM_WORKERS)
_GC_NBUF = 3


def _gconv(h, wg, idxn2):
    mesh = plsc.VectorSubcoreMesh(core_axis_name="c", subcore_axis_name="s")

    @functools.partial(
        pl.kernel, mesh=mesh,
        out_type=jax.ShapeDtypeStruct((_N, _NC), jnp.float32),
        scratch_types=[
            pltpu.VMEM((_MAXB, 1, _EBLK), jnp.int32),
        ] + [pltpu.VMEM((_EBLK, _NC), jnp.float32)] * _GC_NBUF
          + [pltpu.VMEM((_EBLK, _NC // 2), jnp.int32)] * _GC_NBUF
          + [pltpu.VMEM((_NBLK, _NC), jnp.float32)] * _GC_NBUF
          + [pltpu.SemaphoreType.DMA] * (3 * _GC_NBUF),
    )
    def k(h_hbm, wg_hbm, idxn_hbm, out_hbm, idx_v, *rest):
        rows = rest[:_GC_NBUF]
        wbuf = rest[_GC_NBUF:2 * _GC_NBUF]
        obuf = rest[2 * _GC_NBUF:3 * _GC_NBUF]
        gsems = rest[3 * _GC_NBUF:4 * _GC_NBUF]
        wsems = rest[4 * _GC_NBUF:5 * _GC_NBUF]
        osems = rest[5 * _GC_NBUF:]
        wid = lax.axis_index("s") * 2 + lax.axis_index("c")
        blo = (wid * _NBLOCKS) // _NUM_WORKERS
        bhi = ((wid + 1) * _NBLOCKS) // _NUM_WORKERS
        nblk = bhi - blo
        pltpu.sync_copy(idxn_hbm.at[pl.ds(blo, _MAXB)], idx_v)

        def fire(i, p):
            pltpu.async_copy(h_hbm.at[idx_v.at[i, 0]], rows[p], gsems[p])
            pltpu.async_copy(wg_hbm.at[pl.ds((blo + i) * _EBLK, _EBLK)],
                             wbuf[p], wsems[p])

        for q in range(_GC_NBUF):
            @pl.when(q < nblk)
            def _():
                fire(q, q)

        def body(it, carry):
            for p in range(_GC_NBUF):
                i = it * _GC_NBUF + p

                @pl.when(i < nblk)
                def _():
                    pltpu.make_async_copy(
                        h_hbm.at[idx_v.at[i, 0]], rows[p], gsems[p]).wait()
                    pltpu.make_async_copy(
                        wg_hbm.at[pl.ds(0, _EBLK)], wbuf[p], wsems[p]).wait()
                    @pl.when(i >= _GC_NBUF)
                    def _():
                        pltpu.make_async_copy(
                            obuf[p], out_hbm.at[pl.ds(0, _NBLK)],
                            osems[p]).wait()

                    for nn in range(_NBLK):
                        def ebody(j, accs):
                            e = nn * _DEG + j
                            accs = list(accs)
                            for q in range(4):
                                w = wbuf[p][e, pl.ds(16 * q, 16)]
                                wa = lax.bitcast_convert_type(
                                    lax.shift_left(w, 16), jnp.float32)
                                wb = lax.bitcast_convert_type(
                                    jnp.bitwise_and(w, jnp.int32(-65536)),
                                    jnp.float32)
                                ha = rows[p][e, pl.ds(32 * q, 16)]
                                hb = rows[p][e, pl.ds(32 * q + 16, 16)]
                                accs[2 * q] = accs[2 * q] + ha * wa
                                accs[2 * q + 1] = accs[2 * q + 1] + hb * wb
                            return tuple(accs)

                        accs = lax.fori_loop(
                            0, _DEG, ebody,
                            tuple(jnp.zeros((16,), jnp.float32)
                                  for _ in range(_NC // 16)),
                            unroll=4)
                        for c in range(_NC // 16):
                            obuf[p][nn, pl.ds(c * 16, 16)] = accs[c]

                    pltpu.async_copy(
                        obuf[p],
                        out_hbm.at[pl.ds((blo + i) * _NBLK, _NBLK)],
                        osems[p])

                    @pl.when(i + _GC_NBUF < nblk)
                    def _():
                        fire(i + _GC_NBUF, p)

            return carry

        lax.fori_loop(0, -(-_MAXB // _GC_NBUF), body, 0)
        for q in range(_GC_NBUF):
            @pl.when(nblk > q)
            def _():
                pltpu.make_async_copy(obuf[q], out_hbm.at[pl.ds(0, _NBLK)],
                                      osems[q]).wait()

    return k(h, wg, idxn2)


_GB = 1000


def _gru_body(m_ref, s_ref, d_ref, wih_ref, whh_ref, bih_ref, bhh_ref):
    inv = 1.0 / jnp.maximum(d_ref[...].astype(jnp.float32), 1.0)
    x = m_ref[...] * inv
    gi = jnp.dot(x, wih_ref[...],
                 preferred_element_type=jnp.float32) + bih_ref[...]
    gh = jnp.dot(s_ref[...], whh_ref[...],
                 preferred_element_type=jnp.float32) + bhh_ref[...]
    ir, iz, inn = gi[:, :_NC], gi[:, _NC:2 * _NC], gi[:, 2 * _NC:]
    hr, hz, hn = gh[:, :_NC], gh[:, _NC:2 * _NC], gh[:, 2 * _NC:]
    r = jax.nn.sigmoid(ir + hr)
    z = jax.nn.sigmoid(iz + hz)
    n = jnp.tanh(inn + r * hn)
    return (1.0 - z) * n + z * s_ref[...]


_ROW_SPEC = pl.BlockSpec((_GB, _NC), lambda i: (i, 0))
_GRU_IN_SPECS = [
    _ROW_SPEC,
    _ROW_SPEC,
    pl.BlockSpec((_GB, 1), lambda i: (i, 0)),
    pl.BlockSpec((_NC, 3 * _NC), lambda i: (0, 0)),
    pl.BlockSpec((_NC, 3 * _NC), lambda i: (0, 0)),
    pl.BlockSpec((1, 3 * _NC), lambda i: (0, 0)),
    pl.BlockSpec((1, 3 * _NC), lambda i: (0, 0)),
]


def _gru_plain(m, s, degs2, W_ih, W_hh, b_ih2, b_hh2):
    def body(m_ref, s_ref, d_ref, wih_ref, whh_ref, bih_ref, bhh_ref,
             out_ref):
        out_ref[...] = _gru_body(m_ref, s_ref, d_ref, wih_ref, whh_ref,
                                 bih_ref, bhh_ref)

    return pl.pallas_call(
        body,
        grid=(_N // _GB,),
        in_specs=_GRU_IN_SPECS,
        out_specs=_ROW_SPEC,
        out_shape=jax.ShapeDtypeStruct((_N, _NC), jnp.float32),
    )(m, s, degs2, W_ih, W_hh, b_ih2, b_hh2)


def _gru_skip(m, s, add, degs2, W_ih, W_hh, b_ih2, b_hh2):
    def body(m_ref, s_ref, a_ref, d_ref, wih_ref, whh_ref, bih_ref, bhh_ref,
             raw_ref, out_ref):
        raw = _gru_body(m_ref, s_ref, d_ref, wih_ref, whh_ref, bih_ref,
                        bhh_ref)
        raw_ref[...] = raw
        out_ref[...] = raw + a_ref[...]

    return pl.pallas_call(
        body,
        grid=(_N // _GB,),
        in_specs=[_ROW_SPEC, _ROW_SPEC] + _GRU_IN_SPECS[1:],
        out_specs=[_ROW_SPEC, _ROW_SPEC],
        out_shape=[jax.ShapeDtypeStruct((_N, _NC), jnp.float32),
                   jax.ShapeDtypeStruct((_N, _NC), jnp.float32)],
    )(m, s, add, degs2, W_ih, W_hh, b_ih2, b_hh2)


def kernel(hx, edgefeats, idxn, idxe, degs, Wf1, bf1, Wf2, bf2,
           W_ih, W_hh, b_ih, b_hh):
    weights = _filter_net(edgefeats, Wf1, bf1, Wf2, bf2)
    wg = _gather_wg(weights, idxe.reshape(_GA_NBLOCKS, 1, _GA_BLK))
    idxn2 = idxn.reshape(_NBLOCKS, 1, _EBLK)
    degs2 = degs.reshape(_N, 1)
    bih2 = b_ih.reshape(1, 3 * _NC)
    bhh2 = b_hh.reshape(1, 3 * _NC)

    def g_plain(s):
        m = _gconv(s, wg, idxn2)
        return _gru_plain(m, s, degs2, W_ih, W_hh, bih2, bhh2)

    def g_skip(s, add):
        m = _gconv(s, wg, idxn2)
        return _gru_skip(m, s, add, degs2, W_ih, W_hh, bih2, bhh2)

    s1 = g_plain(hx)
    s2 = g_plain(s1)
    r3, s3 = g_skip(s2, s1)
    s4 = g_plain(s3)
    r5, s5 = g_skip(s4, r3)
    s6 = g_plain(s5)
    r7, s7 = g_skip(s6, r5)
    s8 = g_plain(s7)
    _, s9 = g_skip(s8, r7)
    s10 = g_plain(s9)
    return jnp.concatenate(
        [hx, s1, s2, s3, s4, s5, s6, s7, s8, s9, s10], axis=1)
